# baseline (speedup 1.0000x reference)
import functools
import jax
import jax.numpy as jnp
from jax import lax
from jax.experimental import pallas as pl
from jax.experimental.pallas import tpu as pltpu
from jax.experimental.pallas import tpu_sc as plsc

N = 10000
E = 320000
IN_DIM = 512
NDIM = 128
K = 4
DD = NDIM // K
ROUTIT = 6
NLAYER = 4
NCLASS = 40

NC = 2
NS = 16
NW = NC * NS
EW = E // NW
B = 80
NCHUNK = EW // B
NG = B // 16
RPT = 624
RTAIL = N - RPT * NS

MBLK = 1000
GRID = N // MBLK


def _route_body(xn_hbm, cn_hbm, src_hbm, trg_hbm, zer_hbm, out_hbm,
                z_rows, c_rows, src_v, trg_v, acc, sem1, sem2):
    cid = lax.axis_index("c")
    sid = lax.axis_index("s")
    wid = sid * NC + cid

    pltpu.sync_copy(zer_hbm.at[pl.ds(sid * RPT, RPT)],
                    acc.at[pl.ds(sid * RPT, RPT)])

    @pl.when(sid == NS - 1)
    def _():
        pltpu.sync_copy(zer_hbm.at[pl.ds(RPT * NS, RTAIL)],
                        acc.at[pl.ds(RPT * NS, RTAIL)])

    plsc.subcore_barrier()

    def chunk_body(ci, carry):
        ebase = wid * EW + ci * B
        pltpu.sync_copy(src_hbm.at[pl.ds(ebase, B)], src_v)
        pltpu.sync_copy(trg_hbm.at[pl.ds(ebase, B)], trg_v)
        cp1 = pltpu.async_copy(xn_hbm.at[src_v], z_rows, sem1)
        cp2 = pltpu.async_copy(cn_hbm.at[trg_v], c_rows, sem2)
        cp1.wait()
        cp2.wait()

        def group_body(g, gcarry):
            rows = g * 16 + lax.iota(jnp.int32, 16)
            ps = []
            for k in range(K):
                pk = jnp.zeros((16,), jnp.float32)
                for t in range(DD):
                    col = jnp.full((16,), k * DD + t, jnp.int32)
                    zt = plsc.load_gather(z_rows, [rows, col])
                    ct = plsc.load_gather(c_rows, [rows, col])
                    pk = pk + zt * ct
                ps.append(pk)
            m = jnp.maximum(jnp.maximum(ps[0], ps[1]),
                            jnp.maximum(ps[2], ps[3]))
            es = [jnp.exp(p - m) for p in ps]
            ssum = (es[0] + es[1]) + (es[2] + es[3])
            ws = [e / ssum for e in es]
            for k in range(K):
                for t in range(DD):
                    col = jnp.full((16,), k * DD + t, jnp.int32)
                    zt = plsc.load_gather(z_rows, [rows, col])
                    plsc.store_scatter(c_rows, [rows, col], zt * ws[k])
            return gcarry

        lax.fori_loop(0, NG, group_body, 0)
        pltpu.sync_copy(c_rows, acc.at[trg_v], add=True)
        return carry

    lax.fori_loop(0, NCHUNK, chunk_body, 0)
    plsc.subcore_barrier()
    pltpu.sync_copy(acc.at[pl.ds(sid * RPT, RPT)],
                    out_hbm.at[cid, pl.ds(sid * RPT, RPT)])

    @pl.when(sid == NS - 1)
    def _():
        pltpu.sync_copy(acc.at[pl.ds(RPT * NS, RTAIL)],
                        out_hbm.at[cid, pl.ds(RPT * NS, RTAIL)])


_route = pl.kernel(
    _route_body,
    out_type=jax.ShapeDtypeStruct((NC, N, NDIM), jnp.float32),
    mesh=plsc.VectorSubcoreMesh(core_axis_name="c", subcore_axis_name="s"),
    compiler_params=pltpu.CompilerParams(needs_layout_passes=False),
    scratch_types=[
        pltpu.VMEM((B, NDIM), jnp.float32),
        pltpu.VMEM((B, NDIM), jnp.float32),
        pltpu.VMEM((B,), jnp.int32),
        pltpu.VMEM((B,), jnp.int32),
        pltpu.VMEM_SHARED((N, NDIM), jnp.float32),
        pltpu.SemaphoreType.DMA,
        pltpu.SemaphoreType.DMA,
    ],
)


def _capsule_norm(c, mask):
    s = jnp.dot(c * c, mask, preferred_element_type=jnp.float32)
    return c / jnp.maximum(jnp.sqrt(s), 1e-12)


def _pca_body(feat_ref, w_ref, b_ref, mask_ref, o_ref):
    x = jnp.dot(feat_ref[...], w_ref[...], preferred_element_type=jnp.float32)
    x = jnp.maximum(x + b_ref[...], 0.0)
    o_ref[...] = _capsule_norm(x, mask_ref[...])


def _comb_norm_body(c_ref, p0_ref, p1_ref, mask_ref, o_ref):
    c = c_ref[...] + p0_ref[...] + p1_ref[...]
    o_ref[...] = _capsule_norm(c, mask_ref[...])


def _comb_relu_norm_body(c_ref, p0_ref, p1_ref, mask_ref, o_ref):
    c = jnp.maximum(c_ref[...] + p0_ref[...] + p1_ref[...], 0.0)
    o_ref[...] = _capsule_norm(c, mask_ref[...])


def _comb_relu_body(c_ref, p0_ref, p1_ref, o_ref):
    o_ref[...] = jnp.maximum(c_ref[...] + p0_ref[...] + p1_ref[...], 0.0)


def _mlp_body(x_ref, w_ref, b_ref, o_ref):
    logits = jnp.dot(x_ref[...], w_ref[...],
                     preferred_element_type=jnp.float32) + b_ref[...]
    valid = lax.broadcasted_iota(jnp.int32, logits.shape, 1) < NCLASS
    masked = jnp.where(valid, logits, -1e30)
    mx = jnp.max(masked, axis=1, keepdims=True)
    sh = masked - mx
    lse = jnp.log(jnp.sum(jnp.where(valid, jnp.exp(sh), 0.0), axis=1,
                          keepdims=True))
    o_ref[...] = sh - lse


def _row_call(body, full_shapes):
    def make(blocked_cols, out_cols=NDIM):
        in_specs = [pl.BlockSpec((MBLK, c), lambda i: (i, 0))
                    for c in blocked_cols]
        in_specs += [pl.BlockSpec(fs, lambda i: (0, 0)) for fs in full_shapes]
        return pl.pallas_call(
            body,
            grid=(GRID,),
            in_specs=in_specs,
            out_specs=pl.BlockSpec((MBLK, out_cols), lambda i: (i, 0)),
            out_shape=jax.ShapeDtypeStruct((N, out_cols), jnp.float32),
        )
    return make


_pca = _row_call(_pca_body, [(IN_DIM, NDIM), (1, NDIM), (NDIM, NDIM)])([IN_DIM])
_comb_norm = _row_call(_comb_norm_body, [(NDIM, NDIM)])([NDIM, NDIM, NDIM])
_comb_relu_norm = _row_call(_comb_relu_norm_body, [(NDIM, NDIM)])(
    [NDIM, NDIM, NDIM])
_comb_relu = _row_call(_comb_relu_body, [])([NDIM, NDIM, NDIM])
_mlp = _row_call(_mlp_body, [(NDIM, NDIM), (1, NDIM)])([NDIM])


@jax.jit
def kernel(feat, src_trg_edges, W_pca, b_pca, W_mlp, b_mlp):
    src = src_trg_edges[0]
    trg = src_trg_edges[1]
    caps_mask = jnp.kron(jnp.eye(K, dtype=jnp.float32),
                         jnp.ones((DD, DD), jnp.float32))
    zeros_n = jnp.zeros((N, NDIM), jnp.float32)

    xn = _pca(feat, W_pca, b_pca.reshape(1, NDIM), caps_mask)
    for layer in range(NLAYER):
        cn = xn
        for t in range(ROUTIT):
            part = _route(xn, cn, src, trg, zeros_n)
            if t < ROUTIT - 1:
                cn = _comb_norm(cn, part[0], part[1], caps_mask)
            elif layer < NLAYER - 1:
                xn = _comb_relu_norm(cn, part[0], part[1], caps_mask)
            else:
                x_out = _comb_relu(cn, part[0], part[1])

    w_pad = jnp.zeros((NDIM, NDIM), jnp.float32).at[:, :NCLASS].set(W_mlp)
    b_pad = jnp.zeros((1, NDIM), jnp.float32).at[0, :NCLASS].set(b_mlp)
    out = _mlp(x_out, w_pad, b_pad)
    return out[:, :NCLASS]
